# Initial kernel scaffold; baseline (speedup 1.0000x reference)
import jax
import jax.numpy as jnp
from jax.experimental import pallas as pl


def kernel(adj, diff, sub_local_pos1, sub_local_pos2, sub_local_neg1, sub_local_neg2, Wk, bk, Wk1, bk1, Wk2, bk2, alpha, beta, lamda, k):
    raise NotImplementedError("write your pallas kernel here")


# speedup vs baseline: 1.4725x; 1.4725x over previous
import functools

import jax
import jax.numpy as jnp
from jax import lax
from jax.experimental import pallas as pl
from jax.experimental.pallas import tpu as pltpu, tpu_sc as plsc

B, N, D = 4, 4096, 64
DROP = 0.1
BN = 256
KC = 256
RBN = 512
SEG = N // 4

_sc_mesh = plsc.VectorSubcoreMesh(core_axis_name="c", subcore_axis_name="s")
_sc_params = pltpu.CompilerParams(needs_layout_passes=False)


def _rowfold(t):
    acc = t[:, 0:8]
    for c in range(1, 8):
        acc = acc + t[:, 8 * c:8 * c + 8]
    h = 4
    while h >= 1:
        acc = acc[:, :h] + acc[:, h:2 * h]
        h //= 2
    return acc


def _tmp_body(fp_ref, fn_ref, w1_ref, w2_ref, o2p, o2n, o3p, o3n):
    fp = fp_ref[...]
    fn = fn_ref[...]
    w1 = w1_ref[...]
    w2 = w2_ref[...]
    o2p[...] = jnp.dot(fp, w1, preferred_element_type=jnp.float32)
    o2n[...] = jnp.dot(fn, w1, preferred_element_type=jnp.float32)
    o3p[...] = jnp.dot(fp, w2, preferred_element_type=jnp.float32)
    o3n[...] = jnp.dot(fn, w2, preferred_element_type=jnp.float32)


def _tmps(fus_pos, fus_neg, W1, W2):
    sh = jax.ShapeDtypeStruct((B, N, D), jnp.float32)
    return pl.pallas_call(
        _tmp_body,
        grid=(B,),
        in_specs=[pl.BlockSpec((None, N, D), lambda b: (b, 0, 0)),
                  pl.BlockSpec((None, N, D), lambda b: (b, 0, 0)),
                  pl.BlockSpec((D, D), lambda b: (0, 0)),
                  pl.BlockSpec((D, D), lambda b: (0, 0))],
        out_specs=[pl.BlockSpec((None, N, D), lambda b: (b, 0, 0))] * 4,
        out_shape=[sh, sh, sh, sh],
    )(fus_pos, fus_neg, W1, W2)


def _big_body(mat_ref, pos_ref, tp_ref, tn_ref, a_ref, c_ref):
    def step(i, acc):
        return acc + jnp.dot(mat_ref[:, pl.ds(i * KC, KC)],
                             pos_ref[pl.ds(i * KC, KC), :],
                             preferred_element_type=jnp.float32)
    s = lax.fori_loop(0, N // KC, step, jnp.zeros((BN, D), jnp.float32))
    z = jax.nn.sigmoid(s)
    a_ref[...] = _rowfold(tp_ref[...] * z)
    c_ref[...] = _rowfold(tn_ref[...] * z)


def _big(mat, pos, tp, tn):
    sh = jax.ShapeDtypeStruct((B, N, 1), jnp.float32)
    a, c = pl.pallas_call(
        _big_body,
        grid=(B, N // BN),
        in_specs=[pl.BlockSpec((None, BN, N), lambda b, i: (b, i, 0)),
                  pl.BlockSpec((None, N, D), lambda b, i: (b, 0, 0)),
                  pl.BlockSpec((None, BN, D), lambda b, i: (b, i, 0)),
                  pl.BlockSpec((None, BN, D), lambda b, i: (b, i, 0))],
        out_specs=[pl.BlockSpec((None, BN, 1), lambda b, i: (b, i, 0))] * 2,
        out_shape=[sh, sh],
    )(mat, pos, tp, tn)
    return a[..., 0], c[..., 0]


def _l3_body(p1_ref, p2_ref, tp_ref, tn_ref, e1, f1, e2, f2):
    s1 = jax.nn.sigmoid(p1_ref[...])
    s2 = jax.nn.sigmoid(p2_ref[...])
    tp = tp_ref[...]
    tn = tn_ref[...]
    e1[...] = _rowfold(tp * s1)
    f1[...] = _rowfold(tn * s1)
    e2[...] = _rowfold(tp * s2)
    f2[...] = _rowfold(tn * s2)


def _l3(pos1, pos2, tp, tn):
    sh = jax.ShapeDtypeStruct((B, N, 1), jnp.float32)
    outs = pl.pallas_call(
        _l3_body,
        grid=(B,),
        in_specs=[pl.BlockSpec((None, N, D), lambda b: (b, 0, 0))] * 4,
        out_specs=[pl.BlockSpec((None, N, 1), lambda b: (b, 0, 0))] * 4,
        out_shape=[sh] * 4,
    )(pos1, pos2, tp, tn)
    return tuple(o[..., 0] for o in outs)


def _rank_body(col_ref, row_ref, rank_ref):
    ib = pl.program_id(1)
    svc = jax.nn.sigmoid(col_ref[...])
    row = row_ref[...]
    cnt = jnp.zeros((RBN, 1), jnp.float32)
    nblk = N // RBN
    for c in range(nblk):
        svr = jax.nn.sigmoid(row[:, c * RBN:(c + 1) * RBN])
        gtf = jnp.where(svr > svc, 1.0, 0.0)
        geqf = jnp.where(svr >= svc, 1.0, 0.0)
        jg = lax.broadcasted_iota(jnp.int32, (RBN, RBN), 1) + c * RBN
        ig = lax.broadcasted_iota(jnp.int32, (RBN, RBN), 0) + ib * RBN
        diagf = jnp.where(jg < ig, geqf, gtf)
        pred = jnp.where(jnp.int32(c) < ib, geqf,
                         jnp.where(jnp.int32(c) > ib, gtf, diagf))
        cnt = cnt + jnp.sum(pred, axis=1, keepdims=True)
    rank_ref[...] = cnt.astype(jnp.int32)


def _rank(d2g):
    col = d2g.reshape(B, N, 1)
    row = d2g.reshape(B, 1, N)
    r = pl.pallas_call(
        _rank_body,
        grid=(B, N // RBN),
        in_specs=[pl.BlockSpec((None, RBN, 1), lambda b, i: (b, i, 0)),
                  pl.BlockSpec((None, 1, N), lambda b, i: (b, 0, 0))],
        out_specs=pl.BlockSpec((None, RBN, 1), lambda b, i: (b, i, 0)),
        out_shape=jax.ShapeDtypeStruct((B, N, 1), jnp.int32),
    )(col, row)
    return r[..., 0]


@functools.partial(
    pl.kernel, mesh=_sc_mesh, compiler_params=_sc_params,
    out_type=[jax.ShapeDtypeStruct((B, N), jnp.float32),
              jax.ShapeDtypeStruct((B, N), jnp.float32),
              jax.ShapeDtypeStruct((B, N), jnp.float32)],
    scratch_types=[pltpu.VMEM((SEG,), jnp.int32),
                   pltpu.VMEM((N,), jnp.float32),
                   pltpu.VMEM((N,), jnp.float32),
                   pltpu.VMEM((N,), jnp.float32),
                   pltpu.VMEM((SEG,), jnp.float32),
                   pltpu.VMEM((SEG,), jnp.float32),
                   pltpu.VMEM((SEG,), jnp.float32)],
)
def _sc_gather3(gidx, va, vc, vd, outA, outC, outD, idxv, ta, tc, td, oa, oc, od):
    core = lax.axis_index("c")
    s = lax.axis_index("s")
    b = s // 4
    seg = s % 4

    @pl.when(core == 0)
    def _():
        pltpu.sync_copy(gidx.at[b, pl.ds(seg * SEG, SEG)], idxv)
        pltpu.sync_copy(va.at[b], ta)
        pltpu.sync_copy(vc.at[b], tc)
        pltpu.sync_copy(vd.at[b], td)

    plsc.subcore_barrier()

    @pl.when(core == 0)
    def _():
        def body(j, carry):
            iv = idxv[pl.ds(j * 16, 16)]
            oa[pl.ds(j * 16, 16)] = plsc.load_gather(ta, [iv])
            oc[pl.ds(j * 16, 16)] = plsc.load_gather(tc, [iv])
            od[pl.ds(j * 16, 16)] = plsc.load_gather(td, [iv])
            return carry
        lax.fori_loop(0, SEG // 16, body, 0)
        pltpu.sync_copy(oa, outA.at[b, pl.ds(seg * SEG, SEG)])
        pltpu.sync_copy(oc, outC.at[b, pl.ds(seg * SEG, SEG)])
        pltpu.sync_copy(od, outD.at[b, pl.ds(seg * SEG, SEG)])


@functools.partial(
    pl.kernel, mesh=_sc_mesh, compiler_params=_sc_params,
    out_type=jax.ShapeDtypeStruct((B, N), jnp.int32),
    scratch_types=[pltpu.VMEM((N,), jnp.int32),
                   pltpu.VMEM((N,), jnp.int32)],
)
def _sc_scatter(rank, outI, rankv, outv):
    wid = lax.axis_index("s") * 2 + lax.axis_index("c")

    @pl.when(wid < B)
    def _():
        pltpu.sync_copy(rank.at[wid], rankv)

        def body(j, carry):
            rv = rankv[pl.ds(j * 16, 16)]
            vals = lax.iota(jnp.int32, 16) + j * 16
            plsc.store_scatter(outv, [rv], vals)
            return carry
        lax.fori_loop(0, N // 16, body, 0)
        pltpu.sync_copy(outv, outI.at[wid])


@functools.partial(
    pl.kernel, mesh=_sc_mesh, compiler_params=_sc_params,
    out_type=[jax.ShapeDtypeStruct((B, N), jnp.float32),
              jax.ShapeDtypeStruct((B, N), jnp.float32)],
    scratch_types=[pltpu.VMEM((SEG,), jnp.int32),
                   pltpu.VMEM((N,), jnp.int32),
                   pltpu.VMEM((N,), jnp.float32),
                   pltpu.VMEM((N,), jnp.float32),
                   pltpu.VMEM((SEG,), jnp.float32),
                   pltpu.VMEM((SEG,), jnp.float32)],
)
def _sc_compose2(gidx2, gidx, ve, vf, outE, outF, i2v, gv, te, tf, oe, of_):
    core = lax.axis_index("c")
    s = lax.axis_index("s")
    b = s // 4
    seg = s % 4

    @pl.when(core == 0)
    def _():
        pltpu.sync_copy(gidx2.at[b, pl.ds(seg * SEG, SEG)], i2v)
        pltpu.sync_copy(gidx.at[b], gv)
        pltpu.sync_copy(ve.at[b], te)
        pltpu.sync_copy(vf.at[b], tf)

    plsc.subcore_barrier()

    @pl.when(core == 0)
    def _():
        def body(j, carry):
            g2 = i2v[pl.ds(j * 16, 16)]
            ci = plsc.load_gather(gv, [g2])
            oe[pl.ds(j * 16, 16)] = plsc.load_gather(te, [ci])
            of_[pl.ds(j * 16, 16)] = plsc.load_gather(tf, [ci])
            return carry
        lax.fori_loop(0, SEG // 16, body, 0)
        pltpu.sync_copy(oe, outE.at[b, pl.ds(seg * SEG, SEG)])
        pltpu.sync_copy(of_, outF.at[b, pl.ds(seg * SEG, SEG)])


def _bsort4(x):
    a, b, c, d = x[0], x[1], x[2], x[3]
    lo1, hi1 = jnp.minimum(a, b), jnp.maximum(a, b)
    lo2, hi2 = jnp.minimum(c, d), jnp.maximum(c, d)
    r0 = jnp.minimum(lo1, lo2)
    t1 = jnp.maximum(lo1, lo2)
    t2 = jnp.minimum(hi1, hi2)
    r3 = jnp.maximum(hi1, hi2)
    r1 = jnp.minimum(t1, t2)
    r2 = jnp.maximum(t1, t2)
    return jnp.stack((r0, r1, r2, r3), axis=0)


def kernel(adj, diff, sub_local_pos1, sub_local_pos2, sub_local_neg1,
           sub_local_neg2, Wk, bk, Wk1, bk1, Wk2, bk2, alpha, beta, lamda, k):
    rk = jax.random.key(42)
    rk1, rk2 = jax.random.split(rk)
    u1 = jax.random.uniform(rk1, (N, D))
    u2 = jax.random.uniform(rk2, (N, D))
    m1 = u1 < DROP
    m2 = u2 < DROP
    pos1 = jnp.where(m1[None, :, :], 0.0, sub_local_pos1)
    neg1 = jnp.where(m1[None, :, :], 0.0, sub_local_neg1)
    pos2 = jnp.where(m2[None, :, :], 0.0, sub_local_pos2)
    neg2 = jnp.where(m2[None, :, :], 0.0, sub_local_neg2)
    fus_pos = (pos1 + pos2) / 2.0
    fus_neg = (neg1 + neg2) / 2.0

    u1b, u2b, rp1, rn1, rp2, rn2, adjb, diffb = lax.optimization_barrier(
        (u1, u2, sub_local_pos1, sub_local_neg1, sub_local_pos2,
         sub_local_neg2, adj, diff))
    m1b = u1b < DROP
    m2b = u2b < DROP
    pos1b = jnp.where(m1b[None, :, :], 0.0, rp1)
    neg1b = jnp.where(m1b[None, :, :], 0.0, rn1)
    pos2b = jnp.where(m2b[None, :, :], 0.0, rp2)
    neg2b = jnp.where(m2b[None, :, :], 0.0, rn2)
    fus_posb = (pos1b + pos2b) / 2.0
    fus_negb = (neg1b + neg2b) / 2.0

    g1 = jax.nn.sigmoid(jnp.mean(pos1, axis=1))
    g1b = jnp.broadcast_to(g1[:, None, :], pos1.shape)
    g2 = jax.nn.sigmoid(jnp.mean(pos2, axis=1))
    g2b = jnp.broadcast_to(g2[:, None, :], pos2.shape)

    def bil(x1, x2):
        return (jnp.einsum('bni,oij,bnj->bno', x1, Wk, x2) + bk)[..., 0]

    mp1 = bil(fus_pos, g1b)
    mn1 = bil(fus_neg, g1b)
    mp2 = bil(fus_pos, g2b)
    mn2 = bil(fus_neg, g2b)
    lf1 = jnp.concatenate((mp1, mn1), axis=1)
    lf2 = jnp.concatenate((mp2, mn2), axis=1)
    logits_fusion = alpha * lf1 + (1.0 - alpha) * lf2
    score = logits_fusion[:, N:] - logits_fusion[:, :N]
    _, idx_pos = jax.lax.top_k(jax.nn.sigmoid(score), N)
    idx = idx_pos * k
    gidx = _bsort4(idx)

    t2p, t2n, t3p, t3n = _tmps(fus_pos, fus_neg, Wk1[0], Wk2[0])
    a1, c1 = _big(adjb, pos1b, t2p, t2n)
    a2, c2 = _big(diffb, pos2b, t2p, t2n)
    a1, a2, c1, c2 = a1 + bk1, a2 + bk1, c1 + bk1, c2 + bk1
    combA = beta * a1 + (1.0 - beta) * a2
    combC = beta * c1 + (1.0 - beta) * c2
    d2 = combC - combA

    gidx, combA, combC, d2 = lax.optimization_barrier((gidx, combA, combC, d2))
    gA, gC, d2g = _sc_gather3(gidx, combA, combC, d2)
    gA, gC, d2g = lax.optimization_barrier((gA, gC, d2g))
    logits_fusion_sub = jnp.concatenate((gA, gC), axis=1)

    rank2 = _rank(d2g)
    rank2 = lax.optimization_barrier(rank2)
    idx_pos_sub = _sc_scatter(rank2)
    idx_pos_sub = lax.optimization_barrier(idx_pos_sub)
    idx_sub = idx_pos_sub * k
    gidx2 = _bsort4(idx_sub)

    e1, f1, e2, f2 = _l3(pos1b, pos2b, t3p, t3n)
    e1, f1, e2, f2 = e1 + bk2, f1 + bk2, e2 + bk2, f2 + bk2
    combE = lamda * e1 + (1.0 - lamda) * e2
    combF = lamda * f1 + (1.0 - lamda) * f2
    gidx2, gidx, combE, combF = lax.optimization_barrier(
        (gidx2, gidx, combE, combF))
    gE, gF = _sc_compose2(gidx2, gidx, combE, combF)
    gE, gF = lax.optimization_barrier((gE, gF))
    logits_fusion_sub_sub = jnp.concatenate((gE, gF), axis=1)

    return (logits_fusion, logits_fusion_sub, logits_fusion_sub_sub)


# speedup vs baseline: 1.5890x; 1.0792x over previous
import functools

import jax
import jax.numpy as jnp
from jax import lax
from jax.experimental import pallas as pl
from jax.experimental.pallas import tpu as pltpu, tpu_sc as plsc

B, N, D = 4, 4096, 64
DROP = 0.1
BN = 256
KC = 256
RBN = 512
SEG = N // 4

_sc_mesh = plsc.VectorSubcoreMesh(core_axis_name="c", subcore_axis_name="s")
_sc_params = pltpu.CompilerParams(needs_layout_passes=False)


def _rowfold(t):
    acc = t[:, 0:8]
    for c in range(1, 8):
        acc = acc + t[:, 8 * c:8 * c + 8]
    h = 4
    while h >= 1:
        acc = acc[:, :h] + acc[:, h:2 * h]
        h //= 2
    return acc


def _tmp_body(fp_ref, fn_ref, w1_ref, w2_ref, o2p, o2n, o3p, o3n):
    fp = fp_ref[...]
    fn = fn_ref[...]
    w1 = w1_ref[...]
    w2 = w2_ref[...]
    o2p[...] = jnp.dot(fp, w1, preferred_element_type=jnp.float32)
    o2n[...] = jnp.dot(fn, w1, preferred_element_type=jnp.float32)
    o3p[...] = jnp.dot(fp, w2, preferred_element_type=jnp.float32)
    o3n[...] = jnp.dot(fn, w2, preferred_element_type=jnp.float32)


def _tmps(fus_pos, fus_neg, W1, W2):
    sh = jax.ShapeDtypeStruct((B, N, D), jnp.float32)
    return pl.pallas_call(
        _tmp_body,
        grid=(B,),
        in_specs=[pl.BlockSpec((None, N, D), lambda b: (b, 0, 0)),
                  pl.BlockSpec((None, N, D), lambda b: (b, 0, 0)),
                  pl.BlockSpec((D, D), lambda b: (0, 0)),
                  pl.BlockSpec((D, D), lambda b: (0, 0))],
        out_specs=[pl.BlockSpec((None, N, D), lambda b: (b, 0, 0))] * 4,
        out_shape=[sh, sh, sh, sh],
    )(fus_pos, fus_neg, W1, W2)


def _big_body(mat_ref, pos_ref, tp_ref, tn_ref, a_ref, c_ref):
    def step(i, acc):
        return acc + jnp.dot(mat_ref[:, pl.ds(i * KC, KC)],
                             pos_ref[pl.ds(i * KC, KC), :],
                             preferred_element_type=jnp.float32)
    s = lax.fori_loop(0, N // KC, step, jnp.zeros((BN, D), jnp.float32))
    z = jax.nn.sigmoid(s)
    a_ref[...] = _rowfold(tp_ref[...] * z)
    c_ref[...] = _rowfold(tn_ref[...] * z)


def _big(mat, pos, tp, tn):
    sh = jax.ShapeDtypeStruct((B, N, 1), jnp.float32)
    a, c = pl.pallas_call(
        _big_body,
        grid=(B, N // BN),
        in_specs=[pl.BlockSpec((None, BN, N), lambda b, i: (b, i, 0)),
                  pl.BlockSpec((None, N, D), lambda b, i: (b, 0, 0)),
                  pl.BlockSpec((None, BN, D), lambda b, i: (b, i, 0)),
                  pl.BlockSpec((None, BN, D), lambda b, i: (b, i, 0))],
        out_specs=[pl.BlockSpec((None, BN, 1), lambda b, i: (b, i, 0))] * 2,
        out_shape=[sh, sh],
    )(mat, pos, tp, tn)
    return a[..., 0], c[..., 0]


def _l3_body(p1_ref, p2_ref, tp_ref, tn_ref, e1, f1, e2, f2):
    s1 = jax.nn.sigmoid(p1_ref[...])
    s2 = jax.nn.sigmoid(p2_ref[...])
    tp = tp_ref[...]
    tn = tn_ref[...]
    ones = jnp.ones((D, 1), jnp.float32)
    e1[...] = jnp.dot(tp * s1, ones, preferred_element_type=jnp.float32)
    f1[...] = jnp.dot(tn * s1, ones, preferred_element_type=jnp.float32)
    e2[...] = jnp.dot(tp * s2, ones, preferred_element_type=jnp.float32)
    f2[...] = jnp.dot(tn * s2, ones, preferred_element_type=jnp.float32)


def _l3(pos1, pos2, tp, tn):
    sh = jax.ShapeDtypeStruct((B, N, 1), jnp.float32)
    outs = pl.pallas_call(
        _l3_body,
        grid=(B,),
        in_specs=[pl.BlockSpec((None, N, D), lambda b: (b, 0, 0))] * 4,
        out_specs=[pl.BlockSpec((None, N, 1), lambda b: (b, 0, 0))] * 4,
        out_shape=[sh] * 4,
    )(pos1, pos2, tp, tn)
    return tuple(o[..., 0] for o in outs)


def _rank_body(col_ref, row_ref, rank_ref):
    ib = pl.program_id(1)
    svc = jax.nn.sigmoid(col_ref[...])
    row = row_ref[...]
    cnt = jnp.zeros((RBN, 1), jnp.float32)
    nblk = N // RBN
    for c in range(nblk):
        svr = jax.nn.sigmoid(row[:, c * RBN:(c + 1) * RBN])
        gtf = jnp.where(svr > svc, 1.0, 0.0)
        geqf = jnp.where(svr >= svc, 1.0, 0.0)
        jg = lax.broadcasted_iota(jnp.int32, (RBN, RBN), 1) + c * RBN
        ig = lax.broadcasted_iota(jnp.int32, (RBN, RBN), 0) + ib * RBN
        diagf = jnp.where(jg < ig, geqf, gtf)
        pred = jnp.where(jnp.int32(c) < ib, geqf,
                         jnp.where(jnp.int32(c) > ib, gtf, diagf))
        cnt = cnt + jnp.sum(pred, axis=1, keepdims=True)
    rank_ref[...] = cnt.astype(jnp.int32)


def _rank(d2g):
    col = d2g.reshape(B, N, 1)
    row = d2g.reshape(B, 1, N)
    r = pl.pallas_call(
        _rank_body,
        grid=(B, N // RBN),
        in_specs=[pl.BlockSpec((None, RBN, 1), lambda b, i: (b, i, 0)),
                  pl.BlockSpec((None, 1, N), lambda b, i: (b, 0, 0))],
        out_specs=pl.BlockSpec((None, RBN, 1), lambda b, i: (b, i, 0)),
        out_shape=jax.ShapeDtypeStruct((B, N, 1), jnp.int32),
    )(col, row)
    return r[..., 0]


@functools.partial(
    pl.kernel, mesh=_sc_mesh, compiler_params=_sc_params,
    out_type=[jax.ShapeDtypeStruct((B, N), jnp.float32),
              jax.ShapeDtypeStruct((B, N), jnp.float32),
              jax.ShapeDtypeStruct((B, N), jnp.float32)],
    scratch_types=[pltpu.VMEM((SEG,), jnp.int32),
                   pltpu.VMEM((N,), jnp.float32),
                   pltpu.VMEM((N,), jnp.float32),
                   pltpu.VMEM((N,), jnp.float32),
                   pltpu.VMEM((SEG,), jnp.float32),
                   pltpu.VMEM((SEG,), jnp.float32),
                   pltpu.VMEM((SEG,), jnp.float32)],
)
def _sc_gather3(gidx, va, vc, vd, outA, outC, outD, idxv, ta, tc, td, oa, oc, od):
    core = lax.axis_index("c")
    s = lax.axis_index("s")
    b = s // 4
    seg = s % 4

    @pl.when(core == 0)
    def _():
        pltpu.sync_copy(gidx.at[b, pl.ds(seg * SEG, SEG)], idxv)
        pltpu.sync_copy(va.at[b], ta)
        pltpu.sync_copy(vc.at[b], tc)
        pltpu.sync_copy(vd.at[b], td)

    plsc.subcore_barrier()

    @pl.when(core == 0)
    def _():
        def body(j, carry):
            iv = idxv[pl.ds(j * 16, 16)]
            oa[pl.ds(j * 16, 16)] = plsc.load_gather(ta, [iv])
            oc[pl.ds(j * 16, 16)] = plsc.load_gather(tc, [iv])
            od[pl.ds(j * 16, 16)] = plsc.load_gather(td, [iv])
            return carry
        lax.fori_loop(0, SEG // 16, body, 0)
        pltpu.sync_copy(oa, outA.at[b, pl.ds(seg * SEG, SEG)])
        pltpu.sync_copy(oc, outC.at[b, pl.ds(seg * SEG, SEG)])
        pltpu.sync_copy(od, outD.at[b, pl.ds(seg * SEG, SEG)])


@functools.partial(
    pl.kernel, mesh=_sc_mesh, compiler_params=_sc_params,
    out_type=jax.ShapeDtypeStruct((B, N), jnp.int32),
    scratch_types=[pltpu.VMEM((N,), jnp.int32),
                   pltpu.VMEM((N,), jnp.int32)],
)
def _sc_scatter(rank, outI, rankv, outv):
    wid = lax.axis_index("s") * 2 + lax.axis_index("c")

    @pl.when(wid < B)
    def _():
        pltpu.sync_copy(rank.at[wid], rankv)

        def body(j, carry):
            rv = rankv[pl.ds(j * 16, 16)]
            vals = lax.iota(jnp.int32, 16) + j * 16
            plsc.store_scatter(outv, [rv], vals)
            return carry
        lax.fori_loop(0, N // 16, body, 0)
        pltpu.sync_copy(outv, outI.at[wid])


@functools.partial(
    pl.kernel, mesh=_sc_mesh, compiler_params=_sc_params,
    out_type=[jax.ShapeDtypeStruct((B, N), jnp.float32),
              jax.ShapeDtypeStruct((B, N), jnp.float32)],
    scratch_types=[pltpu.VMEM((SEG,), jnp.int32),
                   pltpu.VMEM((N,), jnp.int32),
                   pltpu.VMEM((N,), jnp.float32),
                   pltpu.VMEM((N,), jnp.float32),
                   pltpu.VMEM((SEG,), jnp.float32),
                   pltpu.VMEM((SEG,), jnp.float32)],
)
def _sc_compose2(gidx2, gidx, ve, vf, outE, outF, i2v, gv, te, tf, oe, of_):
    core = lax.axis_index("c")
    s = lax.axis_index("s")
    b = s // 4
    seg = s % 4

    @pl.when(core == 0)
    def _():
        pltpu.sync_copy(gidx2.at[b, pl.ds(seg * SEG, SEG)], i2v)
        pltpu.sync_copy(gidx.at[b], gv)
        pltpu.sync_copy(ve.at[b], te)
        pltpu.sync_copy(vf.at[b], tf)

    plsc.subcore_barrier()

    @pl.when(core == 0)
    def _():
        def body(j, carry):
            g2 = i2v[pl.ds(j * 16, 16)]
            ci = plsc.load_gather(gv, [g2])
            oe[pl.ds(j * 16, 16)] = plsc.load_gather(te, [ci])
            of_[pl.ds(j * 16, 16)] = plsc.load_gather(tf, [ci])
            return carry
        lax.fori_loop(0, SEG // 16, body, 0)
        pltpu.sync_copy(oe, outE.at[b, pl.ds(seg * SEG, SEG)])
        pltpu.sync_copy(of_, outF.at[b, pl.ds(seg * SEG, SEG)])


def _bsort4(x):
    a, b, c, d = x[0], x[1], x[2], x[3]
    lo1, hi1 = jnp.minimum(a, b), jnp.maximum(a, b)
    lo2, hi2 = jnp.minimum(c, d), jnp.maximum(c, d)
    r0 = jnp.minimum(lo1, lo2)
    t1 = jnp.maximum(lo1, lo2)
    t2 = jnp.minimum(hi1, hi2)
    r3 = jnp.maximum(hi1, hi2)
    r1 = jnp.minimum(t1, t2)
    r2 = jnp.maximum(t1, t2)
    return jnp.stack((r0, r1, r2, r3), axis=0)


def kernel(adj, diff, sub_local_pos1, sub_local_pos2, sub_local_neg1,
           sub_local_neg2, Wk, bk, Wk1, bk1, Wk2, bk2, alpha, beta, lamda, k):
    rk = jax.random.key(42)
    rk1, rk2 = jax.random.split(rk)
    u1 = jax.random.uniform(rk1, (N, D))
    u2 = jax.random.uniform(rk2, (N, D))
    m1 = u1 < DROP
    m2 = u2 < DROP
    pos1 = jnp.where(m1[None, :, :], 0.0, sub_local_pos1)
    neg1 = jnp.where(m1[None, :, :], 0.0, sub_local_neg1)
    pos2 = jnp.where(m2[None, :, :], 0.0, sub_local_pos2)
    neg2 = jnp.where(m2[None, :, :], 0.0, sub_local_neg2)
    fus_pos = (pos1 + pos2) / 2.0
    fus_neg = (neg1 + neg2) / 2.0

    u1b, u2b, rp1, rn1, rp2, rn2 = lax.optimization_barrier(
        (u1, u2, sub_local_pos1, sub_local_neg1, sub_local_pos2,
         sub_local_neg2))
    m1b = u1b < DROP
    m2b = u2b < DROP
    pos1b = jnp.where(m1b[None, :, :], 0.0, rp1)
    neg1b = jnp.where(m1b[None, :, :], 0.0, rn1)
    pos2b = jnp.where(m2b[None, :, :], 0.0, rp2)
    neg2b = jnp.where(m2b[None, :, :], 0.0, rn2)
    fus_posb = (pos1b + pos2b) / 2.0
    fus_negb = (neg1b + neg2b) / 2.0

    g1 = jax.nn.sigmoid(jnp.mean(pos1, axis=1))
    g1b = jnp.broadcast_to(g1[:, None, :], pos1.shape)
    g2 = jax.nn.sigmoid(jnp.mean(pos2, axis=1))
    g2b = jnp.broadcast_to(g2[:, None, :], pos2.shape)

    def bil(x1, x2):
        return (jnp.einsum('bni,oij,bnj->bno', x1, Wk, x2) + bk)[..., 0]

    mp1 = bil(fus_pos, g1b)
    mn1 = bil(fus_neg, g1b)
    mp2 = bil(fus_pos, g2b)
    mn2 = bil(fus_neg, g2b)
    lf1 = jnp.concatenate((mp1, mn1), axis=1)
    lf2 = jnp.concatenate((mp2, mn2), axis=1)
    logits_fusion = alpha * lf1 + (1.0 - alpha) * lf2
    score = logits_fusion[:, N:] - logits_fusion[:, :N]
    _, idx_pos = jax.lax.top_k(jax.nn.sigmoid(score), N)
    idx = idx_pos * k
    gidx = _bsort4(idx)

    t2p, t2n, t3p, t3n = _tmps(fus_pos, fus_neg, Wk1[0], Wk2[0])
    a1, c1 = _big(adj, pos1b, t2p, t2n)
    a2, c2 = _big(diff, pos2b, t2p, t2n)
    a1, a2, c1, c2 = a1 + bk1, a2 + bk1, c1 + bk1, c2 + bk1
    combA = beta * a1 + (1.0 - beta) * a2
    combC = beta * c1 + (1.0 - beta) * c2
    d2 = combC - combA

    gidx, combA, combC, d2 = lax.optimization_barrier((gidx, combA, combC, d2))
    gA, gC, d2g = _sc_gather3(gidx, combA, combC, d2)
    gA, gC, d2g = lax.optimization_barrier((gA, gC, d2g))
    logits_fusion_sub = jnp.concatenate((gA, gC), axis=1)

    rank2 = _rank(d2g)
    rank2 = lax.optimization_barrier(rank2)
    idx_pos_sub = _sc_scatter(rank2)
    idx_pos_sub = lax.optimization_barrier(idx_pos_sub)
    idx_sub = idx_pos_sub * k
    gidx2 = _bsort4(idx_sub)

    e1, f1, e2, f2 = _l3(pos1b, pos2b, t3p, t3n)
    e1, f1, e2, f2 = e1 + bk2, f1 + bk2, e2 + bk2, f2 + bk2
    combE = lamda * e1 + (1.0 - lamda) * e2
    combF = lamda * f1 + (1.0 - lamda) * f2
    gidx2, gidx, combE, combF = lax.optimization_barrier(
        (gidx2, gidx, combE, combF))
    gE, gF = _sc_compose2(gidx2, gidx, combE, combF)
    gE, gF = lax.optimization_barrier((gE, gF))
    logits_fusion_sub_sub = jnp.concatenate((gE, gF), axis=1)

    return (logits_fusion, logits_fusion_sub, logits_fusion_sub_sub)


# speedup vs baseline: 1.9782x; 1.2449x over previous
import functools

import jax
import jax.numpy as jnp
from jax import lax
from jax.experimental import pallas as pl
from jax.experimental.pallas import tpu as pltpu, tpu_sc as plsc

B, N, D = 4, 4096, 64
DROP = 0.1
BN = 512
KC = 256
RBN = 512
SEG = N // 4

_sc_mesh = plsc.VectorSubcoreMesh(core_axis_name="c", subcore_axis_name="s")
_sc_params = pltpu.CompilerParams(needs_layout_passes=False)


def _rowfold(t):
    acc = t[:, 0:8]
    for c in range(1, 8):
        acc = acc + t[:, 8 * c:8 * c + 8]
    h = 4
    while h >= 1:
        acc = acc[:, :h] + acc[:, h:2 * h]
        h //= 2
    return acc


def _tmp_body(fp_ref, fn_ref, w1_ref, w2_ref, o2p, o2n, o3p, o3n):
    fp = fp_ref[...]
    fn = fn_ref[...]
    w1 = w1_ref[...]
    w2 = w2_ref[...]
    o2p[...] = jnp.dot(fp, w1, preferred_element_type=jnp.float32)
    o2n[...] = jnp.dot(fn, w1, preferred_element_type=jnp.float32)
    o3p[...] = jnp.dot(fp, w2, preferred_element_type=jnp.float32)
    o3n[...] = jnp.dot(fn, w2, preferred_element_type=jnp.float32)


def _tmps(fus_pos, fus_neg, W1, W2):
    sh = jax.ShapeDtypeStruct((B, N, D), jnp.float32)
    return pl.pallas_call(
        _tmp_body,
        grid=(B,),
        in_specs=[pl.BlockSpec((None, N, D), lambda b: (b, 0, 0)),
                  pl.BlockSpec((None, N, D), lambda b: (b, 0, 0)),
                  pl.BlockSpec((D, D), lambda b: (0, 0)),
                  pl.BlockSpec((D, D), lambda b: (0, 0))],
        out_specs=[pl.BlockSpec((None, N, D), lambda b: (b, 0, 0))] * 4,
        out_shape=[sh, sh, sh, sh],
    )(fus_pos, fus_neg, W1, W2)


def _big_body(mat_ref, pos_ref, tp_ref, tn_ref, a_ref, c_ref):
    def step(i, acc):
        return acc + jnp.dot(mat_ref[:, pl.ds(i * KC, KC)],
                             pos_ref[pl.ds(i * KC, KC), :],
                             preferred_element_type=jnp.float32)
    s = lax.fori_loop(0, N // KC, step, jnp.zeros((BN, D), jnp.float32))
    z = jax.nn.sigmoid(s)
    a_ref[...] = _rowfold(tp_ref[...] * z)
    c_ref[...] = _rowfold(tn_ref[...] * z)


def _big(mat, pos, tp, tn):
    sh = jax.ShapeDtypeStruct((B, N, 1), jnp.float32)
    a, c = pl.pallas_call(
        _big_body,
        grid=(B, N // BN),
        in_specs=[pl.BlockSpec((None, BN, N), lambda b, i: (b, i, 0)),
                  pl.BlockSpec((None, N, D), lambda b, i: (b, 0, 0)),
                  pl.BlockSpec((None, BN, D), lambda b, i: (b, i, 0)),
                  pl.BlockSpec((None, BN, D), lambda b, i: (b, i, 0))],
        out_specs=[pl.BlockSpec((None, BN, 1), lambda b, i: (b, i, 0))] * 2,
        out_shape=[sh, sh],
    )(mat, pos, tp, tn)
    return a[..., 0], c[..., 0]


def _l3_body(p1_ref, p2_ref, tp_ref, tn_ref, e1, f1, e2, f2):
    s1 = jax.nn.sigmoid(p1_ref[...])
    s2 = jax.nn.sigmoid(p2_ref[...])
    tp = tp_ref[...]
    tn = tn_ref[...]
    ones = jnp.ones((D, 1), jnp.float32)
    e1[...] = jnp.dot(tp * s1, ones, preferred_element_type=jnp.float32)
    f1[...] = jnp.dot(tn * s1, ones, preferred_element_type=jnp.float32)
    e2[...] = jnp.dot(tp * s2, ones, preferred_element_type=jnp.float32)
    f2[...] = jnp.dot(tn * s2, ones, preferred_element_type=jnp.float32)


def _l3(pos1, pos2, tp, tn):
    sh = jax.ShapeDtypeStruct((B, N, 1), jnp.float32)
    outs = pl.pallas_call(
        _l3_body,
        grid=(B,),
        in_specs=[pl.BlockSpec((None, N, D), lambda b: (b, 0, 0))] * 4,
        out_specs=[pl.BlockSpec((None, N, 1), lambda b: (b, 0, 0))] * 4,
        out_shape=[sh] * 4,
    )(pos1, pos2, tp, tn)
    return tuple(o[..., 0] for o in outs)


def _rank_body(col_ref, row_ref, rank_ref):
    ib = pl.program_id(1)
    svc = jax.nn.sigmoid(col_ref[...])
    row = row_ref[...]
    cnt = jnp.zeros((RBN, 1), jnp.float32)
    nblk = N // RBN
    for c in range(nblk):
        svr = jax.nn.sigmoid(row[:, c * RBN:(c + 1) * RBN])
        gtf = jnp.where(svr > svc, 1.0, 0.0)
        geqf = jnp.where(svr >= svc, 1.0, 0.0)
        jg = lax.broadcasted_iota(jnp.int32, (RBN, RBN), 1) + c * RBN
        ig = lax.broadcasted_iota(jnp.int32, (RBN, RBN), 0) + ib * RBN
        diagf = jnp.where(jg < ig, geqf, gtf)
        pred = jnp.where(jnp.int32(c) < ib, geqf,
                         jnp.where(jnp.int32(c) > ib, gtf, diagf))
        cnt = cnt + jnp.sum(pred, axis=1, keepdims=True)
    rank_ref[...] = cnt.astype(jnp.int32)


def _rank(d2g):
    col = d2g.reshape(B, N, 1)
    row = d2g.reshape(B, 1, N)
    r = pl.pallas_call(
        _rank_body,
        grid=(B, N // RBN),
        in_specs=[pl.BlockSpec((None, RBN, 1), lambda b, i: (b, i, 0)),
                  pl.BlockSpec((None, 1, N), lambda b, i: (b, 0, 0))],
        out_specs=pl.BlockSpec((None, RBN, 1), lambda b, i: (b, i, 0)),
        out_shape=jax.ShapeDtypeStruct((B, N, 1), jnp.int32),
    )(col, row)
    return r[..., 0]


@functools.partial(
    pl.kernel, mesh=_sc_mesh, compiler_params=_sc_params,
    out_type=[jax.ShapeDtypeStruct((B, N), jnp.float32),
              jax.ShapeDtypeStruct((B, N), jnp.float32),
              jax.ShapeDtypeStruct((B, N), jnp.float32)],
    scratch_types=[pltpu.VMEM((SEG,), jnp.int32),
                   pltpu.VMEM((N,), jnp.float32),
                   pltpu.VMEM((N,), jnp.float32),
                   pltpu.VMEM((N,), jnp.float32),
                   pltpu.VMEM((SEG,), jnp.float32),
                   pltpu.VMEM((SEG,), jnp.float32),
                   pltpu.VMEM((SEG,), jnp.float32)],
)
def _sc_gather3(gidx, va, vc, vd, outA, outC, outD, idxv, ta, tc, td, oa, oc, od):
    core = lax.axis_index("c")
    s = lax.axis_index("s")
    b = s // 4
    seg = s % 4

    @pl.when(core == 0)
    def _():
        pltpu.sync_copy(gidx.at[b, pl.ds(seg * SEG, SEG)], idxv)
        pltpu.sync_copy(va.at[b], ta)
        pltpu.sync_copy(vc.at[b], tc)
        pltpu.sync_copy(vd.at[b], td)

    plsc.subcore_barrier()

    @pl.when(core == 0)
    def _():
        def body(j, carry):
            iv = idxv[pl.ds(j * 16, 16)]
            oa[pl.ds(j * 16, 16)] = plsc.load_gather(ta, [iv])
            oc[pl.ds(j * 16, 16)] = plsc.load_gather(tc, [iv])
            od[pl.ds(j * 16, 16)] = plsc.load_gather(td, [iv])
            return carry
        lax.fori_loop(0, SEG // 16, body, 0)
        pltpu.sync_copy(oa, outA.at[b, pl.ds(seg * SEG, SEG)])
        pltpu.sync_copy(oc, outC.at[b, pl.ds(seg * SEG, SEG)])
        pltpu.sync_copy(od, outD.at[b, pl.ds(seg * SEG, SEG)])


@functools.partial(
    pl.kernel, mesh=_sc_mesh, compiler_params=_sc_params,
    out_type=jax.ShapeDtypeStruct((B, N), jnp.int32),
    scratch_types=[pltpu.VMEM((N,), jnp.int32),
                   pltpu.VMEM((N,), jnp.int32)],
)
def _sc_scatter(rank, outI, rankv, outv):
    wid = lax.axis_index("s") * 2 + lax.axis_index("c")

    @pl.when(wid < B)
    def _():
        pltpu.sync_copy(rank.at[wid], rankv)

        def body(j, carry):
            rv = rankv[pl.ds(j * 16, 16)]
            vals = lax.iota(jnp.int32, 16) + j * 16
            plsc.store_scatter(outv, [rv], vals)
            return carry
        lax.fori_loop(0, N // 16, body, 0)
        pltpu.sync_copy(outv, outI.at[wid])


@functools.partial(
    pl.kernel, mesh=_sc_mesh, compiler_params=_sc_params,
    out_type=[jax.ShapeDtypeStruct((B, N), jnp.float32),
              jax.ShapeDtypeStruct((B, N), jnp.float32)],
    scratch_types=[pltpu.VMEM((SEG,), jnp.int32),
                   pltpu.VMEM((N,), jnp.int32),
                   pltpu.VMEM((N,), jnp.float32),
                   pltpu.VMEM((N,), jnp.float32),
                   pltpu.VMEM((SEG,), jnp.float32),
                   pltpu.VMEM((SEG,), jnp.float32)],
)
def _sc_compose2(gidx2, gidx, ve, vf, outE, outF, i2v, gv, te, tf, oe, of_):
    core = lax.axis_index("c")
    s = lax.axis_index("s")
    b = s // 4
    seg = s % 4

    @pl.when(core == 0)
    def _():
        pltpu.sync_copy(gidx2.at[b, pl.ds(seg * SEG, SEG)], i2v)
        pltpu.sync_copy(gidx.at[b], gv)
        pltpu.sync_copy(ve.at[b], te)
        pltpu.sync_copy(vf.at[b], tf)

    plsc.subcore_barrier()

    @pl.when(core == 0)
    def _():
        def body(j, carry):
            g2 = i2v[pl.ds(j * 16, 16)]
            ci = plsc.load_gather(gv, [g2])
            oe[pl.ds(j * 16, 16)] = plsc.load_gather(te, [ci])
            of_[pl.ds(j * 16, 16)] = plsc.load_gather(tf, [ci])
            return carry
        lax.fori_loop(0, SEG // 16, body, 0)
        pltpu.sync_copy(oe, outE.at[b, pl.ds(seg * SEG, SEG)])
        pltpu.sync_copy(of_, outF.at[b, pl.ds(seg * SEG, SEG)])


def _bsort4(x):
    a, b, c, d = x[0], x[1], x[2], x[3]
    lo1, hi1 = jnp.minimum(a, b), jnp.maximum(a, b)
    lo2, hi2 = jnp.minimum(c, d), jnp.maximum(c, d)
    r0 = jnp.minimum(lo1, lo2)
    t1 = jnp.maximum(lo1, lo2)
    t2 = jnp.minimum(hi1, hi2)
    r3 = jnp.maximum(hi1, hi2)
    r1 = jnp.minimum(t1, t2)
    r2 = jnp.maximum(t1, t2)
    return jnp.stack((r0, r1, r2, r3), axis=0)


def kernel(adj, diff, sub_local_pos1, sub_local_pos2, sub_local_neg1,
           sub_local_neg2, Wk, bk, Wk1, bk1, Wk2, bk2, alpha, beta, lamda, k):
    rk = jax.random.key(42)
    rk1, rk2 = jax.random.split(rk)
    u1 = jax.random.uniform(rk1, (N, D))
    u2 = jax.random.uniform(rk2, (N, D))
    m1 = u1 < DROP
    m2 = u2 < DROP
    pos1 = jnp.where(m1[None, :, :], 0.0, sub_local_pos1)
    neg1 = jnp.where(m1[None, :, :], 0.0, sub_local_neg1)
    pos2 = jnp.where(m2[None, :, :], 0.0, sub_local_pos2)
    neg2 = jnp.where(m2[None, :, :], 0.0, sub_local_neg2)
    fus_pos = (pos1 + pos2) / 2.0
    fus_neg = (neg1 + neg2) / 2.0

    u1b, u2b, rp1, rn1, rp2, rn2 = lax.optimization_barrier(
        (u1, u2, sub_local_pos1, sub_local_neg1, sub_local_pos2,
         sub_local_neg2))
    m1b = u1b < DROP
    m2b = u2b < DROP
    pos1b = jnp.where(m1b[None, :, :], 0.0, rp1)
    neg1b = jnp.where(m1b[None, :, :], 0.0, rn1)
    pos2b = jnp.where(m2b[None, :, :], 0.0, rp2)
    neg2b = jnp.where(m2b[None, :, :], 0.0, rn2)
    fus_posb = (pos1b + pos2b) / 2.0
    fus_negb = (neg1b + neg2b) / 2.0

    g1 = jax.nn.sigmoid(jnp.mean(pos1, axis=1))
    g1b = jnp.broadcast_to(g1[:, None, :], pos1.shape)
    g2 = jax.nn.sigmoid(jnp.mean(pos2, axis=1))
    g2b = jnp.broadcast_to(g2[:, None, :], pos2.shape)

    def bil(x1, x2):
        return (jnp.einsum('bni,oij,bnj->bno', x1, Wk, x2) + bk)[..., 0]

    mp1 = bil(fus_pos, g1b)
    mn1 = bil(fus_neg, g1b)
    mp2 = bil(fus_pos, g2b)
    mn2 = bil(fus_neg, g2b)
    lf1 = jnp.concatenate((mp1, mn1), axis=1)
    lf2 = jnp.concatenate((mp2, mn2), axis=1)
    logits_fusion = alpha * lf1 + (1.0 - alpha) * lf2
    score = logits_fusion[:, N:] - logits_fusion[:, :N]
    _, idx_pos = jax.lax.top_k(jax.nn.sigmoid(score), N)
    idx = idx_pos * k
    gidx = _bsort4(idx)

    t2p, t2n, t3p, t3n = _tmps(fus_pos, fus_neg, Wk1[0], Wk2[0])
    a1, c1 = _big(adj, pos1b, t2p, t2n)
    a2, c2 = _big(diff, pos2b, t2p, t2n)
    a1, a2, c1, c2 = a1 + bk1, a2 + bk1, c1 + bk1, c2 + bk1
    combA = beta * a1 + (1.0 - beta) * a2
    combC = beta * c1 + (1.0 - beta) * c2
    d2 = combC - combA

    gidx, combA, combC, d2 = lax.optimization_barrier((gidx, combA, combC, d2))
    gA, gC, d2g = _sc_gather3(gidx, combA, combC, d2)
    gA, gC, d2g = lax.optimization_barrier((gA, gC, d2g))
    logits_fusion_sub = jnp.concatenate((gA, gC), axis=1)

    rank2 = _rank(d2g)
    rank2 = lax.optimization_barrier(rank2)
    idx_pos_sub = _sc_scatter(rank2)
    idx_pos_sub = lax.optimization_barrier(idx_pos_sub)
    idx_sub = idx_pos_sub * k
    gidx2 = _bsort4(idx_sub)

    e1, f1, e2, f2 = _l3(pos1b, pos2b, t3p, t3n)
    e1, f1, e2, f2 = e1 + bk2, f1 + bk2, e2 + bk2, f2 + bk2
    combE = lamda * e1 + (1.0 - lamda) * e2
    combF = lamda * f1 + (1.0 - lamda) * f2
    gidx2, gidx, combE, combF = lax.optimization_barrier(
        (gidx2, gidx, combE, combF))
    gE, gF = _sc_compose2(gidx2, gidx, combE, combF)
    gE, gF = lax.optimization_barrier((gE, gF))
    logits_fusion_sub_sub = jnp.concatenate((gE, gF), axis=1)

    return (logits_fusion, logits_fusion_sub, logits_fusion_sub_sub)


# speedup vs baseline: 2.2252x; 1.1249x over previous
import functools

import jax
import jax.numpy as jnp
from jax import lax
from jax.experimental import pallas as pl
from jax.experimental.pallas import tpu as pltpu, tpu_sc as plsc

B, N, D = 4, 4096, 64
DROP = 0.1
BN = 1024
KC = 256
RBN = 512
SEG = N // 4

_sc_mesh = plsc.VectorSubcoreMesh(core_axis_name="c", subcore_axis_name="s")
_sc_params = pltpu.CompilerParams(needs_layout_passes=False)


def _rowfold(t):
    acc = t[:, 0:8]
    for c in range(1, 8):
        acc = acc + t[:, 8 * c:8 * c + 8]
    h = 4
    while h >= 1:
        acc = acc[:, :h] + acc[:, h:2 * h]
        h //= 2
    return acc


def _tmp_body(fp_ref, fn_ref, w1_ref, w2_ref, o2p, o2n, o3p, o3n):
    fp = fp_ref[...]
    fn = fn_ref[...]
    w1 = w1_ref[...]
    w2 = w2_ref[...]
    o2p[...] = jnp.dot(fp, w1, preferred_element_type=jnp.float32)
    o2n[...] = jnp.dot(fn, w1, preferred_element_type=jnp.float32)
    o3p[...] = jnp.dot(fp, w2, preferred_element_type=jnp.float32)
    o3n[...] = jnp.dot(fn, w2, preferred_element_type=jnp.float32)


def _tmps(fus_pos, fus_neg, W1, W2):
    sh = jax.ShapeDtypeStruct((B, N, D), jnp.float32)
    return pl.pallas_call(
        _tmp_body,
        grid=(B,),
        in_specs=[pl.BlockSpec((None, N, D), lambda b: (b, 0, 0)),
                  pl.BlockSpec((None, N, D), lambda b: (b, 0, 0)),
                  pl.BlockSpec((D, D), lambda b: (0, 0)),
                  pl.BlockSpec((D, D), lambda b: (0, 0))],
        out_specs=[pl.BlockSpec((None, N, D), lambda b: (b, 0, 0))] * 4,
        out_shape=[sh, sh, sh, sh],
    )(fus_pos, fus_neg, W1, W2)


def _big_body(mat_ref, pos_ref, tp_ref, tn_ref, a_ref, c_ref):
    def step(i, acc):
        return acc + jnp.dot(mat_ref[:, pl.ds(i * KC, KC)],
                             pos_ref[pl.ds(i * KC, KC), :],
                             preferred_element_type=jnp.float32)
    s = lax.fori_loop(0, N // KC, step, jnp.zeros((BN, D), jnp.float32))
    z = jax.nn.sigmoid(s)
    a_ref[...] = _rowfold(tp_ref[...] * z)
    c_ref[...] = _rowfold(tn_ref[...] * z)


def _big(mat, pos, tp, tn):
    sh = jax.ShapeDtypeStruct((B, N, 1), jnp.float32)
    a, c = pl.pallas_call(
        _big_body,
        grid=(B, N // BN),
        in_specs=[pl.BlockSpec((None, BN, N), lambda b, i: (b, i, 0)),
                  pl.BlockSpec((None, N, D), lambda b, i: (b, 0, 0)),
                  pl.BlockSpec((None, BN, D), lambda b, i: (b, i, 0)),
                  pl.BlockSpec((None, BN, D), lambda b, i: (b, i, 0))],
        out_specs=[pl.BlockSpec((None, BN, 1), lambda b, i: (b, i, 0))] * 2,
        out_shape=[sh, sh],
    )(mat, pos, tp, tn)
    return a[..., 0], c[..., 0]


def _l3_body(p1_ref, p2_ref, tp_ref, tn_ref, e1, f1, e2, f2):
    s1 = jax.nn.sigmoid(p1_ref[...])
    s2 = jax.nn.sigmoid(p2_ref[...])
    tp = tp_ref[...]
    tn = tn_ref[...]
    ones = jnp.ones((D, 1), jnp.float32)
    e1[...] = jnp.dot(tp * s1, ones, preferred_element_type=jnp.float32)
    f1[...] = jnp.dot(tn * s1, ones, preferred_element_type=jnp.float32)
    e2[...] = jnp.dot(tp * s2, ones, preferred_element_type=jnp.float32)
    f2[...] = jnp.dot(tn * s2, ones, preferred_element_type=jnp.float32)


def _l3(pos1, pos2, tp, tn):
    sh = jax.ShapeDtypeStruct((B, N, 1), jnp.float32)
    outs = pl.pallas_call(
        _l3_body,
        grid=(B,),
        in_specs=[pl.BlockSpec((None, N, D), lambda b: (b, 0, 0))] * 4,
        out_specs=[pl.BlockSpec((None, N, 1), lambda b: (b, 0, 0))] * 4,
        out_shape=[sh] * 4,
    )(pos1, pos2, tp, tn)
    return tuple(o[..., 0] for o in outs)


def _rank_body(col_ref, row_ref, rank_ref):
    ib = pl.program_id(1)
    svc = jax.nn.sigmoid(col_ref[...])
    row = row_ref[...]
    cnt = jnp.zeros((RBN, 1), jnp.float32)
    nblk = N // RBN
    for c in range(nblk):
        svr = jax.nn.sigmoid(row[:, c * RBN:(c + 1) * RBN])
        gtf = jnp.where(svr > svc, 1.0, 0.0)
        geqf = jnp.where(svr >= svc, 1.0, 0.0)
        jg = lax.broadcasted_iota(jnp.int32, (RBN, RBN), 1) + c * RBN
        ig = lax.broadcasted_iota(jnp.int32, (RBN, RBN), 0) + ib * RBN
        diagf = jnp.where(jg < ig, geqf, gtf)
        pred = jnp.where(jnp.int32(c) < ib, geqf,
                         jnp.where(jnp.int32(c) > ib, gtf, diagf))
        cnt = cnt + jnp.sum(pred, axis=1, keepdims=True)
    rank_ref[...] = cnt.astype(jnp.int32)


def _rank(d2g):
    col = d2g.reshape(B, N, 1)
    row = d2g.reshape(B, 1, N)
    r = pl.pallas_call(
        _rank_body,
        grid=(B, N // RBN),
        in_specs=[pl.BlockSpec((None, RBN, 1), lambda b, i: (b, i, 0)),
                  pl.BlockSpec((None, 1, N), lambda b, i: (b, 0, 0))],
        out_specs=pl.BlockSpec((None, RBN, 1), lambda b, i: (b, i, 0)),
        out_shape=jax.ShapeDtypeStruct((B, N, 1), jnp.int32),
    )(col, row)
    return r[..., 0]


@functools.partial(
    pl.kernel, mesh=_sc_mesh, compiler_params=_sc_params,
    out_type=[jax.ShapeDtypeStruct((B, N), jnp.float32),
              jax.ShapeDtypeStruct((B, N), jnp.float32),
              jax.ShapeDtypeStruct((B, N), jnp.float32)],
    scratch_types=[pltpu.VMEM((SEG,), jnp.int32),
                   pltpu.VMEM((N,), jnp.float32),
                   pltpu.VMEM((N,), jnp.float32),
                   pltpu.VMEM((N,), jnp.float32),
                   pltpu.VMEM((SEG,), jnp.float32),
                   pltpu.VMEM((SEG,), jnp.float32),
                   pltpu.VMEM((SEG,), jnp.float32)],
)
def _sc_gather3(gidx, va, vc, vd, outA, outC, outD, idxv, ta, tc, td, oa, oc, od):
    core = lax.axis_index("c")
    s = lax.axis_index("s")
    b = s // 4
    seg = s % 4

    @pl.when(core == 0)
    def _():
        pltpu.sync_copy(gidx.at[b, pl.ds(seg * SEG, SEG)], idxv)
        pltpu.sync_copy(va.at[b], ta)
        pltpu.sync_copy(vc.at[b], tc)
        pltpu.sync_copy(vd.at[b], td)

    plsc.subcore_barrier()

    @pl.when(core == 0)
    def _():
        def body(j, carry):
            iv = idxv[pl.ds(j * 16, 16)]
            oa[pl.ds(j * 16, 16)] = plsc.load_gather(ta, [iv])
            oc[pl.ds(j * 16, 16)] = plsc.load_gather(tc, [iv])
            od[pl.ds(j * 16, 16)] = plsc.load_gather(td, [iv])
            return carry
        lax.fori_loop(0, SEG // 16, body, 0)
        pltpu.sync_copy(oa, outA.at[b, pl.ds(seg * SEG, SEG)])
        pltpu.sync_copy(oc, outC.at[b, pl.ds(seg * SEG, SEG)])
        pltpu.sync_copy(od, outD.at[b, pl.ds(seg * SEG, SEG)])


@functools.partial(
    pl.kernel, mesh=_sc_mesh, compiler_params=_sc_params,
    out_type=jax.ShapeDtypeStruct((B, N), jnp.int32),
    scratch_types=[pltpu.VMEM((N,), jnp.int32),
                   pltpu.VMEM((N,), jnp.int32)],
)
def _sc_scatter(rank, outI, rankv, outv):
    wid = lax.axis_index("s") * 2 + lax.axis_index("c")

    @pl.when(wid < B)
    def _():
        pltpu.sync_copy(rank.at[wid], rankv)

        def body(j, carry):
            rv = rankv[pl.ds(j * 16, 16)]
            vals = lax.iota(jnp.int32, 16) + j * 16
            plsc.store_scatter(outv, [rv], vals)
            return carry
        lax.fori_loop(0, N // 16, body, 0)
        pltpu.sync_copy(outv, outI.at[wid])


@functools.partial(
    pl.kernel, mesh=_sc_mesh, compiler_params=_sc_params,
    out_type=[jax.ShapeDtypeStruct((B, N), jnp.float32),
              jax.ShapeDtypeStruct((B, N), jnp.float32)],
    scratch_types=[pltpu.VMEM((SEG,), jnp.int32),
                   pltpu.VMEM((N,), jnp.int32),
                   pltpu.VMEM((N,), jnp.float32),
                   pltpu.VMEM((N,), jnp.float32),
                   pltpu.VMEM((SEG,), jnp.float32),
                   pltpu.VMEM((SEG,), jnp.float32)],
)
def _sc_compose2(gidx2, gidx, ve, vf, outE, outF, i2v, gv, te, tf, oe, of_):
    core = lax.axis_index("c")
    s = lax.axis_index("s")
    b = s // 4
    seg = s % 4

    @pl.when(core == 0)
    def _():
        pltpu.sync_copy(gidx2.at[b, pl.ds(seg * SEG, SEG)], i2v)
        pltpu.sync_copy(gidx.at[b], gv)
        pltpu.sync_copy(ve.at[b], te)
        pltpu.sync_copy(vf.at[b], tf)

    plsc.subcore_barrier()

    @pl.when(core == 0)
    def _():
        def body(j, carry):
            g2 = i2v[pl.ds(j * 16, 16)]
            ci = plsc.load_gather(gv, [g2])
            oe[pl.ds(j * 16, 16)] = plsc.load_gather(te, [ci])
            of_[pl.ds(j * 16, 16)] = plsc.load_gather(tf, [ci])
            return carry
        lax.fori_loop(0, SEG // 16, body, 0)
        pltpu.sync_copy(oe, outE.at[b, pl.ds(seg * SEG, SEG)])
        pltpu.sync_copy(of_, outF.at[b, pl.ds(seg * SEG, SEG)])


def _bsort4(x):
    a, b, c, d = x[0], x[1], x[2], x[3]
    lo1, hi1 = jnp.minimum(a, b), jnp.maximum(a, b)
    lo2, hi2 = jnp.minimum(c, d), jnp.maximum(c, d)
    r0 = jnp.minimum(lo1, lo2)
    t1 = jnp.maximum(lo1, lo2)
    t2 = jnp.minimum(hi1, hi2)
    r3 = jnp.maximum(hi1, hi2)
    r1 = jnp.minimum(t1, t2)
    r2 = jnp.maximum(t1, t2)
    return jnp.stack((r0, r1, r2, r3), axis=0)


def kernel(adj, diff, sub_local_pos1, sub_local_pos2, sub_local_neg1,
           sub_local_neg2, Wk, bk, Wk1, bk1, Wk2, bk2, alpha, beta, lamda, k):
    rk = jax.random.key(42)
    rk1, rk2 = jax.random.split(rk)
    u1 = jax.random.uniform(rk1, (N, D))
    u2 = jax.random.uniform(rk2, (N, D))
    m1 = u1 < DROP
    m2 = u2 < DROP
    pos1 = jnp.where(m1[None, :, :], 0.0, sub_local_pos1)
    neg1 = jnp.where(m1[None, :, :], 0.0, sub_local_neg1)
    pos2 = jnp.where(m2[None, :, :], 0.0, sub_local_pos2)
    neg2 = jnp.where(m2[None, :, :], 0.0, sub_local_neg2)
    fus_pos = (pos1 + pos2) / 2.0
    fus_neg = (neg1 + neg2) / 2.0

    u1b, u2b, rp1, rn1, rp2, rn2 = lax.optimization_barrier(
        (u1, u2, sub_local_pos1, sub_local_neg1, sub_local_pos2,
         sub_local_neg2))
    m1b = u1b < DROP
    m2b = u2b < DROP
    pos1b = jnp.where(m1b[None, :, :], 0.0, rp1)
    neg1b = jnp.where(m1b[None, :, :], 0.0, rn1)
    pos2b = jnp.where(m2b[None, :, :], 0.0, rp2)
    neg2b = jnp.where(m2b[None, :, :], 0.0, rn2)
    fus_posb = (pos1b + pos2b) / 2.0
    fus_negb = (neg1b + neg2b) / 2.0

    g1 = jax.nn.sigmoid(jnp.mean(pos1, axis=1))
    g1b = jnp.broadcast_to(g1[:, None, :], pos1.shape)
    g2 = jax.nn.sigmoid(jnp.mean(pos2, axis=1))
    g2b = jnp.broadcast_to(g2[:, None, :], pos2.shape)

    def bil(x1, x2):
        return (jnp.einsum('bni,oij,bnj->bno', x1, Wk, x2) + bk)[..., 0]

    mp1 = bil(fus_pos, g1b)
    mn1 = bil(fus_neg, g1b)
    mp2 = bil(fus_pos, g2b)
    mn2 = bil(fus_neg, g2b)
    lf1 = jnp.concatenate((mp1, mn1), axis=1)
    lf2 = jnp.concatenate((mp2, mn2), axis=1)
    logits_fusion = alpha * lf1 + (1.0 - alpha) * lf2
    score = logits_fusion[:, N:] - logits_fusion[:, :N]
    _, idx_pos = jax.lax.top_k(jax.nn.sigmoid(score), N)
    idx = idx_pos * k
    gidx = _bsort4(idx)

    t2p, t2n, t3p, t3n = _tmps(fus_pos, fus_neg, Wk1[0], Wk2[0])
    a1, c1 = _big(adj, pos1b, t2p, t2n)
    a2, c2 = _big(diff, pos2b, t2p, t2n)
    a1, a2, c1, c2 = a1 + bk1, a2 + bk1, c1 + bk1, c2 + bk1
    combA = beta * a1 + (1.0 - beta) * a2
    combC = beta * c1 + (1.0 - beta) * c2
    d2 = combC - combA

    gidx, combA, combC, d2 = lax.optimization_barrier((gidx, combA, combC, d2))
    gA, gC, d2g = _sc_gather3(gidx, combA, combC, d2)
    gA, gC, d2g = lax.optimization_barrier((gA, gC, d2g))
    logits_fusion_sub = jnp.concatenate((gA, gC), axis=1)

    rank2 = _rank(d2g)
    rank2 = lax.optimization_barrier(rank2)
    idx_pos_sub = _sc_scatter(rank2)
    idx_pos_sub = lax.optimization_barrier(idx_pos_sub)
    idx_sub = idx_pos_sub * k
    gidx2 = _bsort4(idx_sub)

    e1, f1, e2, f2 = _l3(pos1b, pos2b, t3p, t3n)
    e1, f1, e2, f2 = e1 + bk2, f1 + bk2, e2 + bk2, f2 + bk2
    combE = lamda * e1 + (1.0 - lamda) * e2
    combF = lamda * f1 + (1.0 - lamda) * f2
    gidx2, gidx, combE, combF = lax.optimization_barrier(
        (gidx2, gidx, combE, combF))
    gE, gF = _sc_compose2(gidx2, gidx, combE, combF)
    gE, gF = lax.optimization_barrier((gE, gF))
    logits_fusion_sub_sub = jnp.concatenate((gE, gF), axis=1)

    return (logits_fusion, logits_fusion_sub, logits_fusion_sub_sub)
